# baseline (speedup 1.0000x reference)
import functools

import jax
import jax.numpy as jnp
from jax import lax
from jax.experimental import pallas as pl
from jax.experimental.pallas import tpu as pltpu
from jax.experimental.pallas import tpu_sc as plsc

VOCAB = 1000000
PVOCAB = 100000
SVOCAB = 100000
D = 50
C = 5
H = 128
T = 50
B = 16384
DP = 128

NC = 2
NS = 16
NW = NC * NS
N = B * C
NPW = N // NW
G = 128
NG = NPW // G


def _sc_gather(iflat, w2p, w2s, embp, prep, sufp):
    mesh = plsc.VectorSubcoreMesh(core_axis_name="c", subcore_axis_name="s")
    out_sds = jax.ShapeDtypeStruct((N, DP), jnp.float32)

    @functools.partial(
        pl.kernel,
        out_type=(out_sds, out_sds, out_sds),
        mesh=mesh,
        scratch_types=[
            pltpu.VMEM((NG, G), jnp.int32),
            pltpu.VMEM((NG, G), jnp.int32),
            pltpu.VMEM((NG, G), jnp.int32),
            pltpu.VMEM((2, G, DP), jnp.float32),
            pltpu.SemaphoreType.DMA,
            pltpu.SemaphoreType.DMA,
            pltpu.SemaphoreType.DMA,
            pltpu.SemaphoreType.DMA,
        ],
        compiler_params=pltpu.CompilerParams(use_tc_tiling_on_sc=False),
    )
    def body(iflat_hbm, w2p_hbm, w2s_hbm, emb_hbm, pre_hbm, suf_hbm,
             ew_hbm, pw_hbm, sw_hbm,
             idx_all, pidx_all, sidx_all, ebuf,
             semp, sems, sem_e, sem_w):
        wid = lax.axis_index("s") * NC + lax.axis_index("c")
        base = wid * NPW

        pltpu.sync_copy(iflat_hbm.at[wid], idx_all)
        cp_p = []
        cp_s = []
        for g in range(NG):
            cp_p.append(pltpu.async_copy(w2p_hbm.at[idx_all.at[g]],
                                         pidx_all.at[g], semp))
            cp_s.append(pltpu.async_copy(w2s_hbm.at[idx_all.at[g]],
                                         sidx_all.at[g], sems))
        for g in range(NG):
            cp_p[g].wait()
            cp_s[g].wait()

        def run_table(tab_hbm, src_idx, dst_hbm):
            gat = [None] * NG
            wrt = [None] * NG

            def fire(g):
                slot = g % 2
                if g >= 2:
                    wrt[g - 2].wait()
                gat[g] = pltpu.async_copy(tab_hbm.at[src_idx.at[g]],
                                          ebuf.at[slot], sem_e)

            def drain(g):
                slot = g % 2
                gat[g].wait()
                wrt[g] = pltpu.async_copy(ebuf.at[slot],
                                          dst_hbm.at[pl.ds(base + g * G, G)],
                                          sem_w)

            fire(0)
            for g in range(1, NG):
                fire(g)
                drain(g - 1)
            drain(NG - 1)
            wrt[NG - 2].wait()
            wrt[NG - 1].wait()

        run_table(emb_hbm, idx_all, ew_hbm)
        run_table(pre_hbm, pidx_all, pw_hbm)
        run_table(suf_hbm, sidx_all, sw_hbm)

    return body(iflat, w2p, w2s, embp, prep, sufp)


def _mlp_body(ew_ref, pw_ref, sw_ref, w1_ref, b1_ref, w2_ref, b2_ref, out_ref):
    x = ew_ref[...] + pw_ref[...] + sw_ref[...]
    h = b1_ref[...]
    acc = jnp.dot(x[0], w1_ref[0], preferred_element_type=jnp.float32)
    for c in range(1, C):
        acc = acc + jnp.dot(x[c], w1_ref[c],
                            preferred_element_type=jnp.float32)
    h = jnp.tanh(acc + h)
    o = (jnp.dot(h, w2_ref[...], preferred_element_type=jnp.float32)
         + b2_ref[...])
    m = jnp.max(o, axis=1, keepdims=True)
    z = o - m
    lse = jnp.log(jnp.sum(jnp.exp(z), axis=1, keepdims=True))
    out_ref[...] = z - lse


def _mlp(ew, pw, sw, W1c, b1, W2, b2):
    BLK = 2048
    grid = (B // BLK,)
    row_spec = pl.BlockSpec((C, BLK, DP), lambda i: (0, i, 0))
    return pl.pallas_call(
        _mlp_body,
        grid=grid,
        in_specs=[
            row_spec, row_spec, row_spec,
            pl.BlockSpec((C, DP, H), lambda i: (0, 0, 0)),
            pl.BlockSpec((1, H), lambda i: (0, 0)),
            pl.BlockSpec((H, T), lambda i: (0, 0)),
            pl.BlockSpec((1, T), lambda i: (0, 0)),
        ],
        out_specs=pl.BlockSpec((BLK, T), lambda i: (i, 0)),
        out_shape=jax.ShapeDtypeStruct((B, T), jnp.float32),
    )(ew, pw, sw, W1c, b1, W2, b2)


def kernel(inputs, word2prefix, word2suffix, emb, pre_emb, suf_emb,
           W1, b1, W2, b2):
    iflat = inputs.astype(jnp.int32).T.reshape(NW, NG, G)
    embp = jnp.pad(emb, ((0, 0), (0, DP - D)))
    prep = jnp.pad(pre_emb, ((0, 0), (0, DP - D)))
    sufp = jnp.pad(suf_emb, ((0, 0), (0, DP - D)))
    W1c = jnp.pad(W1.reshape(C, D, H), ((0, 0), (0, DP - D), (0, 0)))

    ew, pw, sw = _sc_gather(iflat, word2prefix, word2suffix,
                            embp, prep, sufp)
    ew = ew.reshape(C, B, DP)
    pw = pw.reshape(C, B, DP)
    sw = sw.reshape(C, B, DP)
    return _mlp(ew, pw, sw, W1c, b1.reshape(1, H), W2, b2.reshape(1, T))


# speedup vs baseline: 1.4326x; 1.4326x over previous
import functools

import jax
import jax.numpy as jnp
from jax import lax
from jax.experimental import pallas as pl
from jax.experimental.pallas import tpu as pltpu
from jax.experimental.pallas import tpu_sc as plsc

VOCAB = 1000000
PVOCAB = 100000
SVOCAB = 100000
D = 50
C = 5
H = 128
T = 50
B = 16384
DP = 128

NC = 2
NS = 16
NW = NC * NS
N = B * C
NPW = N // NW
G = 128
NG = NPW // G


def _sc_gather(iflat, w2p, w2s, embp, prep, sufp):
    mesh = plsc.VectorSubcoreMesh(core_axis_name="c", subcore_axis_name="s")
    out_sds = jax.ShapeDtypeStruct((N, DP), jnp.float32)

    @functools.partial(
        pl.kernel,
        out_type=(out_sds, out_sds, out_sds),
        mesh=mesh,
        scratch_types=[
            pltpu.VMEM((NG, G), jnp.int32),
            pltpu.VMEM((NG, G), jnp.int32),
            pltpu.VMEM((NG, G), jnp.int32),
            pltpu.VMEM((2, G, DP), jnp.float32),
            pltpu.SemaphoreType.DMA,
            pltpu.SemaphoreType.DMA,
            pltpu.SemaphoreType.DMA,
            pltpu.SemaphoreType.DMA,
        ],
        compiler_params=pltpu.CompilerParams(use_tc_tiling_on_sc=False),
    )
    def body(iflat_hbm, w2p_hbm, w2s_hbm, emb_hbm, pre_hbm, suf_hbm,
             ew_hbm, pw_hbm, sw_hbm,
             idx_all, pidx_all, sidx_all, ebuf,
             semp, sems, sem_e, sem_w):
        wid = lax.axis_index("s") * NC + lax.axis_index("c")
        base = wid * NPW

        pltpu.sync_copy(iflat_hbm.at[wid], idx_all)
        cp_p = []
        cp_s = []
        for g in range(NG):
            cp_p.append(pltpu.async_copy(w2p_hbm.at[idx_all.at[g]],
                                         pidx_all.at[g], semp))
            cp_s.append(pltpu.async_copy(w2s_hbm.at[idx_all.at[g]],
                                         sidx_all.at[g], sems))
        for g in range(NG):
            cp_p[g].wait()
            cp_s[g].wait()

        def run_table(tab_hbm, src_idx, dst_hbm):
            gat = [None] * NG
            wrt = [None] * NG

            def fire(g):
                slot = g % 2
                if g >= 2:
                    wrt[g - 2].wait()
                gat[g] = pltpu.async_copy(tab_hbm.at[src_idx.at[g]],
                                          ebuf.at[slot], sem_e)

            def drain(g):
                slot = g % 2
                gat[g].wait()
                wrt[g] = pltpu.async_copy(ebuf.at[slot],
                                          dst_hbm.at[pl.ds(base + g * G, G)],
                                          sem_w)

            fire(0)
            for g in range(1, NG):
                fire(g)
                drain(g - 1)
            drain(NG - 1)
            wrt[NG - 2].wait()
            wrt[NG - 1].wait()

        run_table(emb_hbm, idx_all, ew_hbm)
        run_table(pre_hbm, pidx_all, pw_hbm)
        run_table(suf_hbm, sidx_all, sw_hbm)

    return body(iflat, w2p, w2s, embp, prep, sufp)


def _pad_body(in_ref, out_ref):
    out_ref[...] = jnp.pad(in_ref[...], ((0, 0), (0, DP - D)))


def _pad128(t):
    V = t.shape[0]
    BLK = 5000
    return pl.pallas_call(
        _pad_body,
        grid=(V // BLK,),
        in_specs=[pl.BlockSpec((BLK, D), lambda i: (i, 0))],
        out_specs=pl.BlockSpec((BLK, DP), lambda i: (i, 0)),
        out_shape=jax.ShapeDtypeStruct((V, DP), jnp.float32),
    )(t)


def _mlp_body(ew_ref, pw_ref, sw_ref, w1_ref, b1_ref, w2_ref, b2_ref, out_ref):
    x = ew_ref[...] + pw_ref[...] + sw_ref[...]
    h = b1_ref[...]
    acc = jnp.dot(x[0], w1_ref[0], preferred_element_type=jnp.float32)
    for c in range(1, C):
        acc = acc + jnp.dot(x[c], w1_ref[c],
                            preferred_element_type=jnp.float32)
    h = jnp.tanh(acc + h)
    o = (jnp.dot(h, w2_ref[...], preferred_element_type=jnp.float32)
         + b2_ref[...])
    m = jnp.max(o, axis=1, keepdims=True)
    z = o - m
    lse = jnp.log(jnp.sum(jnp.exp(z), axis=1, keepdims=True))
    out_ref[...] = z - lse


def _mlp(ew, pw, sw, W1c, b1, W2, b2):
    BLK = 2048
    grid = (B // BLK,)
    row_spec = pl.BlockSpec((C, BLK, DP), lambda i: (0, i, 0))
    return pl.pallas_call(
        _mlp_body,
        grid=grid,
        in_specs=[
            row_spec, row_spec, row_spec,
            pl.BlockSpec((C, DP, H), lambda i: (0, 0, 0)),
            pl.BlockSpec((1, H), lambda i: (0, 0)),
            pl.BlockSpec((H, T), lambda i: (0, 0)),
            pl.BlockSpec((1, T), lambda i: (0, 0)),
        ],
        out_specs=pl.BlockSpec((BLK, T), lambda i: (i, 0)),
        out_shape=jax.ShapeDtypeStruct((B, T), jnp.float32),
    )(ew, pw, sw, W1c, b1, W2, b2)


def kernel(inputs, word2prefix, word2suffix, emb, pre_emb, suf_emb,
           W1, b1, W2, b2):
    iflat = inputs.astype(jnp.int32).T.reshape(NW, NG, G)
    embp = _pad128(emb)
    prep = _pad128(pre_emb)
    sufp = _pad128(suf_emb)
    W1c = jnp.pad(W1.reshape(C, D, H), ((0, 0), (0, DP - D), (0, 0)))

    ew, pw, sw = _sc_gather(iflat, word2prefix, word2suffix,
                            embp, prep, sufp)
    ew = ew.reshape(C, B, DP)
    pw = pw.reshape(C, B, DP)
    sw = sw.reshape(C, B, DP)
    return _mlp(ew, pw, sw, W1c, b1.reshape(1, H), W2, b2.reshape(1, T))


# speedup vs baseline: 1.5096x; 1.0538x over previous
import functools

import jax
import jax.numpy as jnp
from jax import lax
from jax.experimental import pallas as pl
from jax.experimental.pallas import tpu as pltpu
from jax.experimental.pallas import tpu_sc as plsc

VOCAB = 1000000
PVOCAB = 100000
SVOCAB = 100000
D = 50
C = 5
H = 128
T = 50
B = 16384
DP = 128

NC = 2
NS = 16
NW = NC * NS
N = B * C
NPW = N // NW
G = 128
NG = NPW // G


def _sc_gather(iflat, w2p, w2s, embp, prep, sufp):
    mesh = plsc.VectorSubcoreMesh(core_axis_name="c", subcore_axis_name="s")
    out_sds = jax.ShapeDtypeStruct((N, DP), jnp.float32)

    @functools.partial(
        pl.kernel,
        out_type=(out_sds, out_sds, out_sds),
        mesh=mesh,
        scratch_types=[
            pltpu.VMEM((NG, G), jnp.int32),
            pltpu.VMEM((NG, G), jnp.int32),
            pltpu.VMEM((NG, G), jnp.int32),
            pltpu.VMEM((2, G, DP), jnp.float32),
            pltpu.SemaphoreType.DMA,
            pltpu.SemaphoreType.DMA,
            pltpu.SemaphoreType.DMA,
            pltpu.SemaphoreType.DMA,
        ],
        compiler_params=pltpu.CompilerParams(use_tc_tiling_on_sc=False),
    )
    def body(iflat_hbm, w2p_hbm, w2s_hbm, emb_hbm, pre_hbm, suf_hbm,
             ew_hbm, pw_hbm, sw_hbm,
             idx_all, pidx_all, sidx_all, ebuf,
             semp, sems, sem_e, sem_w):
        wid = lax.axis_index("s") * NC + lax.axis_index("c")
        base = wid * NPW

        pltpu.sync_copy(iflat_hbm.at[wid], idx_all)
        cp_p = []
        cp_s = []
        for g in range(NG):
            cp_p.append(pltpu.async_copy(w2p_hbm.at[idx_all.at[g]],
                                         pidx_all.at[g], semp))
            cp_s.append(pltpu.async_copy(w2s_hbm.at[idx_all.at[g]],
                                         sidx_all.at[g], sems))
        for g in range(NG):
            cp_p[g].wait()
            cp_s[g].wait()

        def run_table(tab_hbm, src_idx, dst_hbm):
            gat = [None] * NG
            wrt = [None] * NG

            def fire(g):
                slot = g % 2
                if g >= 2:
                    wrt[g - 2].wait()
                gat[g] = pltpu.async_copy(tab_hbm.at[src_idx.at[g]],
                                          ebuf.at[slot], sem_e)

            def drain(g):
                slot = g % 2
                gat[g].wait()
                wrt[g] = pltpu.async_copy(ebuf.at[slot],
                                          dst_hbm.at[pl.ds(base + g * G, G)],
                                          sem_w)

            fire(0)
            for g in range(1, NG):
                fire(g)
                drain(g - 1)
            drain(NG - 1)
            wrt[NG - 2].wait()
            wrt[NG - 1].wait()

        run_table(emb_hbm, idx_all, ew_hbm)
        run_table(pre_hbm, pidx_all, pw_hbm)
        run_table(suf_hbm, sidx_all, sw_hbm)

    return body(iflat, w2p, w2s, embp, prep, sufp)


def _pad_body(in_ref, out_ref):
    out_ref[...] = jnp.pad(in_ref[...], ((0, 0), (0, DP - D)))


def _pad128(t):
    V = t.shape[0]
    BLK = 5000
    return pl.pallas_call(
        _pad_body,
        grid=(V // BLK,),
        in_specs=[pl.BlockSpec((BLK, D), lambda i: (i, 0))],
        out_specs=pl.BlockSpec((BLK, DP), lambda i: (i, 0)),
        out_shape=jax.ShapeDtypeStruct((V, DP), jnp.float32),
    )(t)


def _mlp_body(ew_ref, pw_ref, sw_ref, w1_ref, b1_ref, w2_ref, b2_ref, out_ref):
    x = ew_ref[...] + pw_ref[...] + sw_ref[...]
    h = b1_ref[...]
    acc = jnp.dot(x[0], w1_ref[0], preferred_element_type=jnp.float32)
    for c in range(1, C):
        acc = acc + jnp.dot(x[c], w1_ref[c],
                            preferred_element_type=jnp.float32)
    h = jnp.tanh(acc + h)
    o = (jnp.dot(h, w2_ref[...], preferred_element_type=jnp.float32)
         + b2_ref[...])
    m = jnp.max(o, axis=1, keepdims=True)
    z = o - m
    lse = jnp.log(jnp.sum(jnp.exp(z), axis=1, keepdims=True))
    out_ref[...] = z - lse


def _mlp(ew, pw, sw, W1c, b1, W2, b2):
    BLK = 2048
    grid = (B // BLK,)
    row_spec = pl.BlockSpec((C, BLK, DP), lambda i: (0, i, 0))
    return pl.pallas_call(
        _mlp_body,
        grid=grid,
        in_specs=[
            row_spec, row_spec, row_spec,
            pl.BlockSpec((C, DP, H), lambda i: (0, 0, 0)),
            pl.BlockSpec((1, H), lambda i: (0, 0)),
            pl.BlockSpec((H, T), lambda i: (0, 0)),
            pl.BlockSpec((1, T), lambda i: (0, 0)),
        ],
        out_specs=pl.BlockSpec((BLK, T), lambda i: (i, 0)),
        out_shape=jax.ShapeDtypeStruct((B, T), jnp.float32),
    )(ew, pw, sw, W1c, b1, W2, b2)


def kernel(inputs, word2prefix, word2suffix, emb, pre_emb, suf_emb,
           W1, b1, W2, b2):
    iflat = inputs.astype(jnp.int32).T.reshape(NW, NG, G)
    embp = _pad128(emb)
    prep = jnp.pad(pre_emb, ((0, 0), (0, DP - D)))
    sufp = jnp.pad(suf_emb, ((0, 0), (0, DP - D)))
    W1c = jnp.pad(W1.reshape(C, D, H), ((0, 0), (0, DP - D), (0, 0)))

    ew, pw, sw = _sc_gather(iflat, word2prefix, word2suffix,
                            embp, prep, sufp)
    ew = ew.reshape(C, B, DP)
    pw = pw.reshape(C, B, DP)
    sw = sw.reshape(C, B, DP)
    return _mlp(ew, pw, sw, W1c, b1.reshape(1, H), W2, b2.reshape(1, T))


# speedup vs baseline: 1.5336x; 1.0159x over previous
import functools

import jax
import jax.numpy as jnp
from jax import lax
from jax.experimental import pallas as pl
from jax.experimental.pallas import tpu as pltpu
from jax.experimental.pallas import tpu_sc as plsc

VOCAB = 1000000
PVOCAB = 100000
SVOCAB = 100000
D = 50
C = 5
H = 128
T = 50
B = 16384
DP = 128

NC = 2
NS = 16
NW = NC * NS
N = B * C
NPW = N // NW
G = 128
NG = NPW // G


def _sc_gather(iflat, w2p, w2s, embp, prep, sufp):
    mesh = plsc.VectorSubcoreMesh(core_axis_name="c", subcore_axis_name="s")
    out_sds = jax.ShapeDtypeStruct((N, DP), jnp.float32)

    @functools.partial(
        pl.kernel,
        out_type=(out_sds, out_sds, out_sds),
        mesh=mesh,
        scratch_types=[
            pltpu.VMEM((NG, G), jnp.int32),
            pltpu.VMEM((NG, G), jnp.int32),
            pltpu.VMEM((NG, G), jnp.int32),
            pltpu.VMEM((2, G, DP), jnp.float32),
            pltpu.SemaphoreType.DMA,
            pltpu.SemaphoreType.DMA,
            pltpu.SemaphoreType.DMA,
            pltpu.SemaphoreType.DMA,
        ],
        compiler_params=pltpu.CompilerParams(use_tc_tiling_on_sc=False),
    )
    def body(iflat_hbm, w2p_hbm, w2s_hbm, emb_hbm, pre_hbm, suf_hbm,
             ew_hbm, pw_hbm, sw_hbm,
             idx_all, pidx_all, sidx_all, ebuf,
             semp, sems, sem_e, sem_w):
        wid = lax.axis_index("s") * NC + lax.axis_index("c")
        base = wid * NPW

        pltpu.sync_copy(iflat_hbm.at[wid], idx_all)
        cp_p = []
        cp_s = []
        for g in range(NG):
            cp_p.append(pltpu.async_copy(w2p_hbm.at[idx_all.at[g]],
                                         pidx_all.at[g], semp))
            cp_s.append(pltpu.async_copy(w2s_hbm.at[idx_all.at[g]],
                                         sidx_all.at[g], sems))
        for g in range(NG):
            cp_p[g].wait()
            cp_s[g].wait()

        def run_table(tab_hbm, src_idx, dst_hbm):
            gat = [None] * NG
            wrt = [None] * NG

            def fire(g):
                slot = g % 2
                if g >= 2:
                    wrt[g - 2].wait()
                gat[g] = pltpu.async_copy(tab_hbm.at[src_idx.at[g]],
                                          ebuf.at[slot], sem_e)

            def drain(g):
                slot = g % 2
                gat[g].wait()
                wrt[g] = pltpu.async_copy(ebuf.at[slot],
                                          dst_hbm.at[pl.ds(base + g * G, G)],
                                          sem_w)

            fire(0)
            for g in range(1, NG):
                fire(g)
                drain(g - 1)
            drain(NG - 1)
            wrt[NG - 2].wait()
            wrt[NG - 1].wait()

        run_table(emb_hbm, idx_all, ew_hbm)
        run_table(pre_hbm, pidx_all, pw_hbm)
        run_table(suf_hbm, sidx_all, sw_hbm)

    return body(iflat, w2p, w2s, embp, prep, sufp)


def _pad_body(in_ref, out_ref):
    out_ref[...] = jnp.pad(in_ref[...], ((0, 0), (0, DP - D)))


def _pad128(t):
    V = t.shape[0]
    BLK = 25000
    return pl.pallas_call(
        _pad_body,
        grid=(V // BLK,),
        in_specs=[pl.BlockSpec((BLK, D), lambda i: (i, 0))],
        out_specs=pl.BlockSpec((BLK, DP), lambda i: (i, 0)),
        out_shape=jax.ShapeDtypeStruct((V, DP), jnp.float32),
    )(t)


def _mlp_body(ew_ref, pw_ref, sw_ref, w1_ref, b1_ref, w2_ref, b2_ref, out_ref):
    x = ew_ref[...] + pw_ref[...] + sw_ref[...]
    h = b1_ref[...]
    acc = jnp.dot(x[0], w1_ref[0], preferred_element_type=jnp.float32)
    for c in range(1, C):
        acc = acc + jnp.dot(x[c], w1_ref[c],
                            preferred_element_type=jnp.float32)
    h = jnp.tanh(acc + h)
    o = (jnp.dot(h, w2_ref[...], preferred_element_type=jnp.float32)
         + b2_ref[...])
    m = jnp.max(o, axis=1, keepdims=True)
    z = o - m
    lse = jnp.log(jnp.sum(jnp.exp(z), axis=1, keepdims=True))
    out_ref[...] = z - lse


def _mlp(ew, pw, sw, W1c, b1, W2, b2):
    BLK = 2048
    grid = (B // BLK,)
    row_spec = pl.BlockSpec((C, BLK, DP), lambda i: (0, i, 0))
    return pl.pallas_call(
        _mlp_body,
        grid=grid,
        in_specs=[
            row_spec, row_spec, row_spec,
            pl.BlockSpec((C, DP, H), lambda i: (0, 0, 0)),
            pl.BlockSpec((1, H), lambda i: (0, 0)),
            pl.BlockSpec((H, T), lambda i: (0, 0)),
            pl.BlockSpec((1, T), lambda i: (0, 0)),
        ],
        out_specs=pl.BlockSpec((BLK, T), lambda i: (i, 0)),
        out_shape=jax.ShapeDtypeStruct((B, T), jnp.float32),
    )(ew, pw, sw, W1c, b1, W2, b2)


def kernel(inputs, word2prefix, word2suffix, emb, pre_emb, suf_emb,
           W1, b1, W2, b2):
    iflat = inputs.astype(jnp.int32).T.reshape(NW, NG, G)
    embp = _pad128(emb)
    prep = jnp.pad(pre_emb, ((0, 0), (0, DP - D)))
    sufp = jnp.pad(suf_emb, ((0, 0), (0, DP - D)))
    W1c = jnp.pad(W1.reshape(C, D, H), ((0, 0), (0, DP - D), (0, 0)))

    ew, pw, sw = _sc_gather(iflat, word2prefix, word2suffix,
                            embp, prep, sufp)
    ew = ew.reshape(C, B, DP)
    pw = pw.reshape(C, B, DP)
    sw = sw.reshape(C, B, DP)
    return _mlp(ew, pw, sw, W1c, b1.reshape(1, H), W2, b2.reshape(1, T))
